# baseline (speedup 1.0000x reference)
import functools

import jax
import jax.numpy as jnp
import numpy as np
from jax import lax
from jax.experimental import pallas as pl
from jax.experimental.pallas import tpu as pltpu
from jax.experimental.pallas import tpu_sc as plsc

NUM_STEPS = 1000
BETA_START = 0.0001
BETA_END = 0.02

_beta = np.linspace(BETA_START, BETA_END, NUM_STEPS, dtype=np.float32)
_alpha_bar = np.cumprod((1.0 - _beta).astype(np.float64))
_SQRT_AB = np.sqrt(_alpha_bar).astype(np.float32)
_SQRT_1MAB = np.sqrt(1.0 - _alpha_bar).astype(np.float32)


def _sc_gather(t, sa_tab, sv_tab):
    b = t.shape[0]
    n_tab = sa_tab.shape[0]
    lanes = 16
    mesh = plsc.VectorSubcoreMesh(core_axis_name="c", subcore_axis_name="s")

    @functools.partial(
        pl.kernel,
        out_type=(
            jax.ShapeDtypeStruct((b,), jnp.float32),
            jax.ShapeDtypeStruct((b,), jnp.float32),
        ),
        mesh=mesh,
        compiler_params=pltpu.CompilerParams(needs_layout_passes=False),
        scratch_types=[
            pltpu.VMEM((b,), jnp.int32),
            pltpu.VMEM((n_tab,), jnp.float32),
            pltpu.VMEM((n_tab,), jnp.float32),
            pltpu.VMEM((b,), jnp.float32),
            pltpu.VMEM((b,), jnp.float32),
        ],
    )
    def k(t_hbm, sa_hbm, sv_hbm, sa_out, sv_out, idx_v, sa_tab_v, sv_tab_v, sa_v, sv_v):
        @pl.when((lax.axis_index("c") == 0) & (lax.axis_index("s") == 0))
        def _():
            pltpu.sync_copy(t_hbm, idx_v)
            pltpu.sync_copy(sa_hbm, sa_tab_v)
            pltpu.sync_copy(sv_hbm, sv_tab_v)
            for g in range(b // lanes):
                ti = idx_v[pl.ds(g * lanes, lanes)]
                sa_v[pl.ds(g * lanes, lanes)] = plsc.load_gather(sa_tab_v, [ti])
                sv_v[pl.ds(g * lanes, lanes)] = plsc.load_gather(sv_tab_v, [ti])
            pltpu.sync_copy(sa_v, sa_out)
            pltpu.sync_copy(sv_v, sv_out)

    return k(t, sa_tab, sv_tab)


def _tc_body_coeff(sa_ref, sv_ref, x_ref, e_ref, o_ref):
    i = pl.program_id(0)
    n = x_ref.shape[0]
    for j in range(n):
        a = sa_ref[i * n + j]
        v = sv_ref[i * n + j]
        o_ref[j] = a * x_ref[j] + v * e_ref[j]


def _tc_fma_coeff(sa, sv, x, e, bs):
    b, c, h, w = x.shape
    grid = (b // bs,)
    blk = pl.BlockSpec((bs, c, h, w), lambda i: (i, 0, 0, 0))
    return pl.pallas_call(
        _tc_body_coeff,
        grid=grid,
        in_specs=[
            pl.BlockSpec(memory_space=pltpu.SMEM),
            pl.BlockSpec(memory_space=pltpu.SMEM),
            blk,
            blk,
        ],
        out_specs=blk,
        out_shape=jax.ShapeDtypeStruct((b, c, h, w), jnp.float32),
        compiler_params=pltpu.CompilerParams(
            dimension_semantics=("parallel",),
        ),
    )(sa, sv, x, e)


def _tc_body(t_ref, sa_ref, sv_ref, x_ref, e_ref, o_ref):
    i = pl.program_id(0)
    n = x_ref.shape[0]
    for j in range(n):
        tt = t_ref[i * n + j]
        a = sa_ref[tt]
        v = sv_ref[tt]
        o_ref[j] = a * x_ref[j] + v * e_ref[j]


def _tc_fma(t, sa_tab, sv_tab, x, e, bs):
    b, c, h, w = x.shape
    grid = (b // bs,)
    blk = pl.BlockSpec((bs, c, h, w), lambda i: (i, 0, 0, 0))
    return pl.pallas_call(
        _tc_body,
        grid=grid,
        in_specs=[
            pl.BlockSpec(memory_space=pltpu.SMEM),
            pl.BlockSpec(memory_space=pltpu.SMEM),
            pl.BlockSpec(memory_space=pltpu.SMEM),
            blk,
            blk,
        ],
        out_specs=blk,
        out_shape=jax.ShapeDtypeStruct((b, c, h, w), jnp.float32),
        compiler_params=pltpu.CompilerParams(
            dimension_semantics=("parallel",),
        ),
    )(t, sa_tab, sv_tab, x, e)


def kernel(x0, t, eps):
    t32 = t.astype(jnp.int32)
    sa_t, sv_t = _sc_gather(t32, jnp.asarray(_SQRT_AB), jnp.asarray(_SQRT_1MAB))
    return _tc_fma_coeff(sa_t, sv_t, x0, eps, bs=1)


# speedup vs baseline: 1.2313x; 1.2313x over previous
import jax
import jax.numpy as jnp
import numpy as np
from jax.experimental import pallas as pl
from jax.experimental.pallas import tpu as pltpu

NUM_STEPS = 1000
BETA_START = 0.0001
BETA_END = 0.02

_beta = np.linspace(BETA_START, BETA_END, NUM_STEPS, dtype=np.float32)
_alpha_bar = np.cumprod((1.0 - _beta).astype(np.float64))
_SQRT_AB = np.sqrt(_alpha_bar).astype(np.float32)
_SQRT_1MAB = np.sqrt(1.0 - _alpha_bar).astype(np.float32)


def _tc_body(t_ref, sa_ref, sv_ref, x_ref, e_ref, o_ref):
    i = pl.program_id(0)
    n = x_ref.shape[0]
    for j in range(n):
        tt = t_ref[i * n + j]
        a = sa_ref[tt]
        v = sv_ref[tt]
        o_ref[j] = a * x_ref[j] + v * e_ref[j]


def _tc_fma(t, sa_tab, sv_tab, x, e, bs):
    b, c, h, w = x.shape
    grid = (b // bs,)
    blk = pl.BlockSpec((bs, c, h, w), lambda i: (i, 0, 0, 0))
    return pl.pallas_call(
        _tc_body,
        grid=grid,
        in_specs=[
            pl.BlockSpec(memory_space=pltpu.SMEM),
            pl.BlockSpec(memory_space=pltpu.SMEM),
            pl.BlockSpec(memory_space=pltpu.SMEM),
            blk,
            blk,
        ],
        out_specs=blk,
        out_shape=jax.ShapeDtypeStruct((b, c, h, w), jnp.float32),
        compiler_params=pltpu.CompilerParams(
            dimension_semantics=("parallel",),
        ),
    )(t, sa_tab, sv_tab, x, e)


def kernel(x0, t, eps):
    t32 = t.astype(jnp.int32)
    return _tc_fma(t32, jnp.asarray(_SQRT_AB), jnp.asarray(_SQRT_1MAB),
                   x0, eps, bs=1)
